# baseline (speedup 1.0000x reference)
import functools

import jax
import jax.numpy as jnp
from jax.experimental import pallas as pl
from jax.experimental.pallas import tpu as pltpu

_EPS = 1e-10
_N_TOK = 128
_VOCAB = 100000
_CHUNK = 4096
_NCHUNK = (_VOCAB + _CHUNK - 1) // _CHUNK


def _argmax_body(logits_ref, temps_ref, exp_ref, out_ref, bestv_ref, besti_ref):
    j = pl.program_id(0)
    x = logits_ref[...] / temps_ref[...]
    g = -jnp.log(exp_ref[...])
    s = x + g
    col = jax.lax.broadcasted_iota(jnp.int32, s.shape, 1) + j * _CHUNK
    s = jnp.where(col < _VOCAB, s, -jnp.inf)
    bmax = jnp.max(s, axis=1, keepdims=True)
    barg = (jnp.argmax(s, axis=1).astype(jnp.int32) + j * _CHUNK)[:, None]

    @pl.when(j == 0)
    def _():
        bestv_ref[...] = bmax
        besti_ref[...] = barg

    @pl.when(j > 0)
    def _():
        upd = bmax > bestv_ref[...]
        bestv_ref[...] = jnp.where(upd, bmax, bestv_ref[...])
        besti_ref[...] = jnp.where(upd, barg, besti_ref[...])

    @pl.when(j == _NCHUNK - 1)
    def _():
        out_ref[...] = besti_ref[...]


@functools.partial(jax.jit, static_argnames=())
def kernel(logits, temperatures, exponential):
    temps = jnp.clip(temperatures, _EPS, None).reshape(_N_TOK, 1)
    out = pl.pallas_call(
        _argmax_body,
        grid=(_NCHUNK,),
        in_specs=[
            pl.BlockSpec((_N_TOK, _CHUNK), lambda j: (0, j)),
            pl.BlockSpec((_N_TOK, 1), lambda j: (0, 0)),
            pl.BlockSpec((1, _CHUNK), lambda j: (0, j)),
        ],
        out_specs=pl.BlockSpec((_N_TOK, 1), lambda j: (0, 0)),
        out_shape=jax.ShapeDtypeStruct((_N_TOK, 1), jnp.int32),
        scratch_shapes=[
            pltpu.VMEM((_N_TOK, 1), jnp.float32),
            pltpu.VMEM((_N_TOK, 1), jnp.int32),
        ],
    )(logits, temps, exponential)
    return out.reshape(_N_TOK)


# speedup vs baseline: 1.0783x; 1.0783x over previous
import functools

import jax
import jax.numpy as jnp
from jax.experimental import pallas as pl
from jax.experimental.pallas import tpu as pltpu

_EPS = 1e-10
_N_TOK = 128
_VOCAB = 100000
_LANE = 128
_CHUNK = 8192
_GROUPS = _CHUNK // _LANE
_NCHUNK = (_VOCAB + _CHUNK - 1) // _CHUNK
_BIG = 2**30


def _scan_groups(s, j):
    m = s[:, :_LANE]
    a = jnp.zeros((_N_TOK, _LANE), jnp.int32)
    for g in range(1, _GROUPS):
        blk = s[:, g * _LANE:(g + 1) * _LANE]
        upd = blk > m
        m = jnp.where(upd, blk, m)
        a = jnp.where(upd, jnp.int32(g), a)
    return m, a + j * _GROUPS


def _body(logits_ref, invt_ref, gum_ref, out_ref, accv_ref, accg_ref):
    j = pl.program_id(0)
    s = logits_ref[...] * invt_ref[...] + gum_ref[...]

    def _merge(m, a):
        upd = m > accv_ref[...]
        accv_ref[...] = jnp.where(upd, m, accv_ref[...])
        accg_ref[...] = jnp.where(upd, a, accg_ref[...])

    @pl.when(j == 0)
    def _():
        m, a = _scan_groups(s, j)
        accv_ref[...] = m
        accg_ref[...] = a

    @pl.when(jnp.logical_and(j > 0, j < _NCHUNK - 1))
    def _():
        _merge(*_scan_groups(s, j))

    @pl.when(j == _NCHUNK - 1)
    def _():
        col = jax.lax.broadcasted_iota(jnp.int32, s.shape, 1) + j * _CHUNK
        _merge(*_scan_groups(jnp.where(col < _VOCAB, s, -jnp.inf), j))
        accv = accv_ref[...]
        best = accv.max(axis=1, keepdims=True)
        lane = jax.lax.broadcasted_iota(jnp.int32, accv.shape, 1)
        cand = accg_ref[...] * _LANE + lane
        out_ref[...] = jnp.min(
            jnp.where(accv == best, cand, _BIG), axis=1, keepdims=True
        )


@functools.partial(jax.jit, static_argnames=())
def kernel(logits, temperatures, exponential):
    invt = (1.0 / jnp.clip(temperatures, _EPS, None)).reshape(_N_TOK, 1)
    gum = -jnp.log(exponential)
    out = pl.pallas_call(
        _body,
        grid=(_NCHUNK,),
        in_specs=[
            pl.BlockSpec((_N_TOK, _CHUNK), lambda j: (0, j)),
            pl.BlockSpec((_N_TOK, 1), lambda j: (0, 0)),
            pl.BlockSpec((1, _CHUNK), lambda j: (0, j)),
        ],
        out_specs=pl.BlockSpec((_N_TOK, 1), lambda j: (0, 0)),
        out_shape=jax.ShapeDtypeStruct((_N_TOK, 1), jnp.int32),
        scratch_shapes=[
            pltpu.VMEM((_N_TOK, _LANE), jnp.float32),
            pltpu.VMEM((_N_TOK, _LANE), jnp.int32),
        ],
    )(logits, invt, gum)
    return out.reshape(_N_TOK)
